# baseline (speedup 1.0000x reference)
import functools
import jax
import jax.numpy as jnp
from jax import lax
from jax.experimental import pallas as pl
from jax.experimental.pallas import tpu as pltpu

_K = 32
_NSHELLS = 3
_RADII = (0.2, 0.4, 0.8)
_SCALE = 0.69314718056 * _NSHELLS ** 2


def _kth_mask_T(d2T, k):
    bits = lax.bitcast_convert_type(d2T, jnp.int32)
    nt = d2T.shape[1]
    lo0 = jnp.zeros((1, nt), jnp.int32)
    hi0 = jnp.full((1, nt), jnp.int32(0x7F7FFFFF))

    def body(_, carry):
        lo, hi = carry
        mid = (lo + hi) >> 1
        cnt = jnp.sum((bits <= mid).astype(jnp.int32), axis=0, keepdims=True)
        pred = cnt >= k
        return jnp.where(pred, lo, mid + 1), jnp.where(pred, mid, hi)

    lo, hi = lax.fori_loop(0, 31, body, (lo0, hi0), unroll=8)
    return (bits <= lo).astype(jnp.float32)


_DN0 = (((0,), (0,)), ((), ()))


def _tfn_layer(src, tgt, tgtT, feats, Wc, bc, Wa, ba, Wb, bb, radius):
    dpx = src[:, 0:1] - tgtT[0:1, :]
    dpy = src[:, 1:2] - tgtT[1:2, :]
    dpz = src[:, 2:3] - tgtT[2:3, :]
    d2 = dpx * dpx + dpy * dpy + dpz * dpz
    mask = _kth_mask_T(d2, _K)
    dist = jnp.sqrt(d2 + 1e-12)
    inv = 1.0 / (dist + 1e-8)
    planes = []
    for s in range(_NSHELLS):
        c = s / (_NSHELLS - 1.0)
        g = jnp.exp(-_SCALE * (dist / radius - c) ** 2) * mask
        gi = g * inv
        planes.extend([g, gi * dpx, gi * dpy, gi * dpz])
    cols = [lax.dot_general(p, feats, _DN0, preferred_element_type=jnp.float32)
            for p in planes]
    y = jnp.concatenate(cols, axis=1) * (1.0 / _K)
    y = jnp.maximum(jnp.dot(y, Wc, preferred_element_type=jnp.float32) + bc, 0.0)
    y = jnp.maximum(jnp.dot(y, Wa, preferred_element_type=jnp.float32) + ba, 0.0)
    y = jnp.maximum(jnp.dot(y, Wb, preferred_element_type=jnp.float32) + bb, 0.0)
    return y


def _body(x_ref, p1_ref, p1T_ref, p2_ref, p2T_ref, p3_ref, p3T_ref,
          Wc0_ref, bc0_ref, Wm0a_ref, bm0a_ref, Wm0b_ref, bm0b_ref,
          Wc1_ref, bc1_ref, Wm1a_ref, bm1a_ref, Wm1b_ref, bm1b_ref,
          Wc2_ref, bc2_ref, Wm2a_ref, bm2a_ref, Wm2b_ref, bm2b_ref,
          Wfc1_ref, bfc1_ref, Wfc2_ref, bfc2_ref, Wout_ref, bout_ref,
          out_ref):
    x = x_ref[0]
    p1, p1T = p1_ref[0], p1T_ref[0]
    p2, p2T = p2_ref[0], p2T_ref[0]
    p3, p3T = p3_ref[0], p3T_ref[0]

    f = _tfn_layer(x, p1, p1T, x, Wc0_ref[...], bc0_ref[...],
                   Wm0a_ref[...], bm0a_ref[...], Wm0b_ref[...], bm0b_ref[...],
                   _RADII[0])
    f = _tfn_layer(p1, p2, p2T, f, Wc1_ref[...], bc1_ref[...],
                   Wm1a_ref[...], bm1a_ref[...], Wm1b_ref[...], bm1b_ref[...],
                   _RADII[1])
    f = _tfn_layer(p2, p3, p3T, f, Wc2_ref[...], bc2_ref[...],
                   Wm2a_ref[...], bm2a_ref[...], Wm2b_ref[...], bm2b_ref[...],
                   _RADII[2])
    g = jnp.max(f, axis=0, keepdims=True)
    h = jnp.maximum(jnp.dot(g, Wfc1_ref[...], preferred_element_type=jnp.float32) + bfc1_ref[...], 0.0)
    h = jnp.maximum(jnp.dot(h, Wfc2_ref[...], preferred_element_type=jnp.float32) + bfc2_ref[...], 0.0)
    logits = jnp.dot(h, Wout_ref[...], preferred_element_type=jnp.float32) + bout_ref[...]
    m = jnp.max(logits, axis=-1, keepdims=True)
    e = jnp.exp(logits - m)
    out_ref[0] = e / jnp.sum(e, axis=-1, keepdims=True)


@jax.jit
def kernel(x, Wc0, bc0, Wm0a, bm0a, Wm0b, bm0b,
           Wc1, bc1, Wm1a, bm1a, Wm1b, bm1b,
           Wc2, bc2, Wm2a, bm2a, Wm2b, bm2b,
           Wfc1, bfc1, Wfc2, bfc2, Wout, bout):
    B, N, _ = x.shape
    xT = jnp.swapaxes(x, 1, 2)
    p1 = x[:, ::4, :]
    p1T = xT[:, :, ::4]
    p2 = x[:, ::16, :]
    p2T = xT[:, :, ::16]
    p3 = x[:, ::64, :]
    p3T = xT[:, :, ::64]

    def pb(a):
        return pl.BlockSpec((1,) + a.shape[1:], lambda b: (b,) + (0,) * (a.ndim - 1))

    def full(a):
        return pl.BlockSpec(a.shape, lambda b: (0,) * a.ndim)

    r2 = lambda b: b.reshape(1, -1)
    weights = (Wc0, r2(bc0), Wm0a, r2(bm0a), Wm0b, r2(bm0b),
               Wc1, r2(bc1), Wm1a, r2(bm1a), Wm1b, r2(bm1b),
               Wc2, r2(bc2), Wm2a, r2(bm2a), Wm2b, r2(bm2b),
               Wfc1, r2(bfc1), Wfc2, r2(bfc2), Wout, r2(bout))
    operands = (x, p1, p1T, p2, p2T, p3, p3T) + weights
    in_specs = [pb(x), pb(p1), pb(p1T), pb(p2), pb(p2T), pb(p3), pb(p3T)] + \
               [full(w) for w in weights]

    out = pl.pallas_call(
        _body,
        grid=(B,),
        in_specs=in_specs,
        out_specs=pl.BlockSpec((1, 1, 40), lambda b: (b, 0, 0)),
        out_shape=jax.ShapeDtypeStruct((B, 1, 40), jnp.float32),
        compiler_params=pltpu.CompilerParams(
            dimension_semantics=("arbitrary",),
        ),
    )(*operands)
    return out.reshape(B, 40)


# speedup vs baseline: 1.0877x; 1.0877x over previous
import functools
import jax
import jax.numpy as jnp
from jax import lax
from jax.experimental import pallas as pl
from jax.experimental.pallas import tpu as pltpu

_K = 32
_NSHELLS = 3
_RADII = (0.2, 0.4, 0.8)
_SCALE = 0.69314718056 * _NSHELLS ** 2


def _kth_mask_T(d2T, k):
    bits = lax.bitcast_convert_type(d2T, jnp.int32)
    nt = d2T.shape[1]
    h16 = ((bits >> 15) - 32768).astype(jnp.int16)
    lo0 = jnp.zeros((1, nt), jnp.int32)
    hi0 = jnp.full((1, nt), jnp.int32(65535))

    def body16(_, carry):
        lo, hi = carry
        mid = (lo + hi) >> 1
        mid16 = (mid - 32768).astype(jnp.int16)
        x = (h16 <= mid16).astype(jnp.int16)
        n = x.shape[0]
        while n > 1:
            n //= 2
            x = x[:n] + x[n:]
        pred = x.astype(jnp.int32) >= k
        return jnp.where(pred, lo, mid + 1), jnp.where(pred, mid, hi)

    lo, hi = lax.fori_loop(0, 16, body16, (lo0, hi0), unroll=4)
    lo = lo << 15
    hi = lo + 32767

    def body(_, carry):
        lo, hi = carry
        mid = (lo + hi) >> 1
        cnt = jnp.sum((bits <= mid).astype(jnp.int32), axis=0, keepdims=True)
        pred = cnt >= k
        return jnp.where(pred, lo, mid + 1), jnp.where(pred, mid, hi)

    lo, hi = lax.fori_loop(0, 15, body, (lo, hi), unroll=4)
    return (bits <= lo).astype(jnp.float32)


_DN0 = (((0,), (0,)), ((), ()))


def _tfn_layer(src, tgt, tgtT, feats, Wc, bc, Wa, ba, Wb, bb, radius):
    dpx = src[:, 0:1] - tgtT[0:1, :]
    dpy = src[:, 1:2] - tgtT[1:2, :]
    dpz = src[:, 2:3] - tgtT[2:3, :]
    d2 = dpx * dpx + dpy * dpy + dpz * dpz
    mask = _kth_mask_T(d2, _K)
    dist = jnp.sqrt(d2 + 1e-12)
    inv = 1.0 / (dist + 1e-8)
    planes = []
    for s in range(_NSHELLS):
        c = s / (_NSHELLS - 1.0)
        g = jnp.exp(-_SCALE * (dist / radius - c) ** 2) * mask
        gi = g * inv
        planes.extend([g, gi * dpx, gi * dpy, gi * dpz])
    cols = [lax.dot_general(p, feats, _DN0, preferred_element_type=jnp.float32)
            for p in planes]
    y = jnp.concatenate(cols, axis=1) * (1.0 / _K)
    y = jnp.maximum(jnp.dot(y, Wc, preferred_element_type=jnp.float32) + bc, 0.0)
    y = jnp.maximum(jnp.dot(y, Wa, preferred_element_type=jnp.float32) + ba, 0.0)
    y = jnp.maximum(jnp.dot(y, Wb, preferred_element_type=jnp.float32) + bb, 0.0)
    return y


def _body(x_ref, p1_ref, p1T_ref, p2_ref, p2T_ref, p3_ref, p3T_ref,
          Wc0_ref, bc0_ref, Wm0a_ref, bm0a_ref, Wm0b_ref, bm0b_ref,
          Wc1_ref, bc1_ref, Wm1a_ref, bm1a_ref, Wm1b_ref, bm1b_ref,
          Wc2_ref, bc2_ref, Wm2a_ref, bm2a_ref, Wm2b_ref, bm2b_ref,
          Wfc1_ref, bfc1_ref, Wfc2_ref, bfc2_ref, Wout_ref, bout_ref,
          out_ref):
    x = x_ref[0]
    p1, p1T = p1_ref[0], p1T_ref[0]
    p2, p2T = p2_ref[0], p2T_ref[0]
    p3, p3T = p3_ref[0], p3T_ref[0]

    f = _tfn_layer(x, p1, p1T, x, Wc0_ref[...], bc0_ref[...],
                   Wm0a_ref[...], bm0a_ref[...], Wm0b_ref[...], bm0b_ref[...],
                   _RADII[0])
    f = _tfn_layer(p1, p2, p2T, f, Wc1_ref[...], bc1_ref[...],
                   Wm1a_ref[...], bm1a_ref[...], Wm1b_ref[...], bm1b_ref[...],
                   _RADII[1])
    f = _tfn_layer(p2, p3, p3T, f, Wc2_ref[...], bc2_ref[...],
                   Wm2a_ref[...], bm2a_ref[...], Wm2b_ref[...], bm2b_ref[...],
                   _RADII[2])
    g = jnp.max(f, axis=0, keepdims=True)
    h = jnp.maximum(jnp.dot(g, Wfc1_ref[...], preferred_element_type=jnp.float32) + bfc1_ref[...], 0.0)
    h = jnp.maximum(jnp.dot(h, Wfc2_ref[...], preferred_element_type=jnp.float32) + bfc2_ref[...], 0.0)
    logits = jnp.dot(h, Wout_ref[...], preferred_element_type=jnp.float32) + bout_ref[...]
    m = jnp.max(logits, axis=-1, keepdims=True)
    e = jnp.exp(logits - m)
    out_ref[0] = e / jnp.sum(e, axis=-1, keepdims=True)


@jax.jit
def kernel(x, Wc0, bc0, Wm0a, bm0a, Wm0b, bm0b,
           Wc1, bc1, Wm1a, bm1a, Wm1b, bm1b,
           Wc2, bc2, Wm2a, bm2a, Wm2b, bm2b,
           Wfc1, bfc1, Wfc2, bfc2, Wout, bout):
    B, N, _ = x.shape
    xT = jnp.swapaxes(x, 1, 2)
    p1 = x[:, ::4, :]
    p1T = xT[:, :, ::4]
    p2 = x[:, ::16, :]
    p2T = xT[:, :, ::16]
    p3 = x[:, ::64, :]
    p3T = xT[:, :, ::64]

    def pb(a):
        return pl.BlockSpec((1,) + a.shape[1:], lambda b: (b,) + (0,) * (a.ndim - 1))

    def full(a):
        return pl.BlockSpec(a.shape, lambda b: (0,) * a.ndim)

    r2 = lambda b: b.reshape(1, -1)
    weights = (Wc0, r2(bc0), Wm0a, r2(bm0a), Wm0b, r2(bm0b),
               Wc1, r2(bc1), Wm1a, r2(bm1a), Wm1b, r2(bm1b),
               Wc2, r2(bc2), Wm2a, r2(bm2a), Wm2b, r2(bm2b),
               Wfc1, r2(bfc1), Wfc2, r2(bfc2), Wout, r2(bout))
    operands = (x, p1, p1T, p2, p2T, p3, p3T) + weights
    in_specs = [pb(x), pb(p1), pb(p1T), pb(p2), pb(p2T), pb(p3), pb(p3T)] + \
               [full(w) for w in weights]

    out = pl.pallas_call(
        _body,
        grid=(B,),
        in_specs=in_specs,
        out_specs=pl.BlockSpec((1, 1, 40), lambda b: (b, 0, 0)),
        out_shape=jax.ShapeDtypeStruct((B, 1, 40), jnp.float32),
        compiler_params=pltpu.CompilerParams(
            dimension_semantics=("arbitrary",),
        ),
    )(*operands)
    return out.reshape(B, 40)


# speedup vs baseline: 1.1696x; 1.0753x over previous
import functools
import jax
import jax.numpy as jnp
from jax import lax
from jax.experimental import pallas as pl
from jax.experimental.pallas import tpu as pltpu

_K = 32
_NSHELLS = 3
_RADII = (0.2, 0.4, 0.8)
_SCALE = 0.69314718056 * _NSHELLS ** 2


def _kth_mask_T(d2T, k):
    bits = lax.bitcast_convert_type(d2T, jnp.int32)
    nt = d2T.shape[1]
    h16 = ((bits >> 15) - 32768).astype(jnp.int16)
    lo0 = jnp.zeros((1, nt), jnp.int32)
    hi0 = jnp.full((1, nt), jnp.int32(65535))

    def body16(_, carry):
        lo, hi = carry
        mid = (lo + hi) >> 1
        mid16 = (mid - 32768).astype(jnp.int16)
        x = (h16 <= mid16).astype(jnp.int16)
        n = x.shape[0]
        while n > 1:
            n //= 2
            x = x[:n] + x[n:]
        pred = x.astype(jnp.int32) >= k
        return jnp.where(pred, lo, mid + 1), jnp.where(pred, mid, hi)

    T, _ = lax.fori_loop(0, 16, body16, (lo0, hi0), unroll=4)
    T16 = (T - 32768).astype(jnp.int16)
    clt = (h16 < T16).astype(jnp.int16)
    n = clt.shape[0]
    while n > 1:
        n //= 2
        clt = clt[:n] + clt[n:]
    clt = clt.astype(jnp.int32)
    l15 = jnp.where(h16 == T16, (bits & 0x7FFF).astype(jnp.int16),
                    jnp.int16(32767))

    def body15(_, carry):
        lo, hi = carry
        mid = (lo + hi) >> 1
        x = (l15 <= mid.astype(jnp.int16)).astype(jnp.int16)
        n = x.shape[0]
        while n > 1:
            n //= 2
            x = x[:n] + x[n:]
        pred = clt + x.astype(jnp.int32) >= k
        return jnp.where(pred, lo, mid + 1), jnp.where(pred, mid, hi)

    lo, hi = lax.fori_loop(0, 15, body15,
                           (jnp.zeros((1, nt), jnp.int32),
                            jnp.full((1, nt), jnp.int32(32767))), unroll=4)
    thr = (T << 15) | lo
    return (bits <= thr).astype(jnp.float32)


_DN0 = (((0,), (0,)), ((), ()))


def _tfn_layer(src, tgt, tgtT, feats, Wc, bc, Wa, ba, Wb, bb, radius):
    dpx = src[:, 0:1] - tgtT[0:1, :]
    dpy = src[:, 1:2] - tgtT[1:2, :]
    dpz = src[:, 2:3] - tgtT[2:3, :]
    d2 = dpx * dpx + dpy * dpy + dpz * dpz
    mask = _kth_mask_T(d2, _K)
    dist = jnp.sqrt(d2 + 1e-12)
    inv = 1.0 / (dist + 1e-8)
    planes = []
    for s in range(_NSHELLS):
        c = s / (_NSHELLS - 1.0)
        g = jnp.exp(-_SCALE * (dist / radius - c) ** 2) * mask
        gi = g * inv
        planes.extend([g, gi * dpx, gi * dpy, gi * dpz])
    cols = [lax.dot_general(p, feats, _DN0, preferred_element_type=jnp.float32)
            for p in planes]
    y = jnp.concatenate(cols, axis=1) * (1.0 / _K)
    y = jnp.maximum(jnp.dot(y, Wc, preferred_element_type=jnp.float32) + bc, 0.0)
    y = jnp.maximum(jnp.dot(y, Wa, preferred_element_type=jnp.float32) + ba, 0.0)
    y = jnp.maximum(jnp.dot(y, Wb, preferred_element_type=jnp.float32) + bb, 0.0)
    return y


def _body(x_ref, p1_ref, p1T_ref, p2_ref, p2T_ref, p3_ref, p3T_ref,
          Wc0_ref, bc0_ref, Wm0a_ref, bm0a_ref, Wm0b_ref, bm0b_ref,
          Wc1_ref, bc1_ref, Wm1a_ref, bm1a_ref, Wm1b_ref, bm1b_ref,
          Wc2_ref, bc2_ref, Wm2a_ref, bm2a_ref, Wm2b_ref, bm2b_ref,
          Wfc1_ref, bfc1_ref, Wfc2_ref, bfc2_ref, Wout_ref, bout_ref,
          out_ref):
    x = x_ref[0]
    p1, p1T = p1_ref[0], p1T_ref[0]
    p2, p2T = p2_ref[0], p2T_ref[0]
    p3, p3T = p3_ref[0], p3T_ref[0]

    f = _tfn_layer(x, p1, p1T, x, Wc0_ref[...], bc0_ref[...],
                   Wm0a_ref[...], bm0a_ref[...], Wm0b_ref[...], bm0b_ref[...],
                   _RADII[0])
    f = _tfn_layer(p1, p2, p2T, f, Wc1_ref[...], bc1_ref[...],
                   Wm1a_ref[...], bm1a_ref[...], Wm1b_ref[...], bm1b_ref[...],
                   _RADII[1])
    f = _tfn_layer(p2, p3, p3T, f, Wc2_ref[...], bc2_ref[...],
                   Wm2a_ref[...], bm2a_ref[...], Wm2b_ref[...], bm2b_ref[...],
                   _RADII[2])
    g = jnp.max(f, axis=0, keepdims=True)
    h = jnp.maximum(jnp.dot(g, Wfc1_ref[...], preferred_element_type=jnp.float32) + bfc1_ref[...], 0.0)
    h = jnp.maximum(jnp.dot(h, Wfc2_ref[...], preferred_element_type=jnp.float32) + bfc2_ref[...], 0.0)
    logits = jnp.dot(h, Wout_ref[...], preferred_element_type=jnp.float32) + bout_ref[...]
    m = jnp.max(logits, axis=-1, keepdims=True)
    e = jnp.exp(logits - m)
    out_ref[0] = e / jnp.sum(e, axis=-1, keepdims=True)


@jax.jit
def kernel(x, Wc0, bc0, Wm0a, bm0a, Wm0b, bm0b,
           Wc1, bc1, Wm1a, bm1a, Wm1b, bm1b,
           Wc2, bc2, Wm2a, bm2a, Wm2b, bm2b,
           Wfc1, bfc1, Wfc2, bfc2, Wout, bout):
    B, N, _ = x.shape
    xT = jnp.swapaxes(x, 1, 2)
    p1 = x[:, ::4, :]
    p1T = xT[:, :, ::4]
    p2 = x[:, ::16, :]
    p2T = xT[:, :, ::16]
    p3 = x[:, ::64, :]
    p3T = xT[:, :, ::64]

    def pb(a):
        return pl.BlockSpec((1,) + a.shape[1:], lambda b: (b,) + (0,) * (a.ndim - 1))

    def full(a):
        return pl.BlockSpec(a.shape, lambda b: (0,) * a.ndim)

    r2 = lambda b: b.reshape(1, -1)
    weights = (Wc0, r2(bc0), Wm0a, r2(bm0a), Wm0b, r2(bm0b),
               Wc1, r2(bc1), Wm1a, r2(bm1a), Wm1b, r2(bm1b),
               Wc2, r2(bc2), Wm2a, r2(bm2a), Wm2b, r2(bm2b),
               Wfc1, r2(bfc1), Wfc2, r2(bfc2), Wout, r2(bout))
    operands = (x, p1, p1T, p2, p2T, p3, p3T) + weights
    in_specs = [pb(x), pb(p1), pb(p1T), pb(p2), pb(p2T), pb(p3), pb(p3T)] + \
               [full(w) for w in weights]

    out = pl.pallas_call(
        _body,
        grid=(B,),
        in_specs=in_specs,
        out_specs=pl.BlockSpec((1, 1, 40), lambda b: (b, 0, 0)),
        out_shape=jax.ShapeDtypeStruct((B, 1, 40), jnp.float32),
        compiler_params=pltpu.CompilerParams(
            dimension_semantics=("arbitrary",),
        ),
    )(*operands)
    return out.reshape(B, 40)


# speedup vs baseline: 1.1814x; 1.0101x over previous
import functools
import jax
import jax.numpy as jnp
from jax import lax
from jax.experimental import pallas as pl
from jax.experimental.pallas import tpu as pltpu

_K = 32
_NSHELLS = 3
_RADII = (0.2, 0.4, 0.8)
_SCALE = 0.69314718056 * _NSHELLS ** 2


def _kth_mask_T(d2T, k):
    bits = lax.bitcast_convert_type(d2T, jnp.int32)
    nt = d2T.shape[1]
    h16 = ((bits >> 15) - 32768).astype(jnp.int16)
    lo0 = jnp.zeros((1, nt), jnp.int32)
    hi0 = jnp.full((1, nt), jnp.int32(65535))

    def body16(_, carry):
        lo, hi = carry
        mid = (lo + hi) >> 1
        mid16 = (mid - 32768).astype(jnp.int16)
        x = (h16 <= mid16).astype(jnp.int16)
        n = x.shape[0]
        while n > 1:
            n //= 2
            x = x[:n] + x[n:]
        pred = x.astype(jnp.int32) >= k
        return jnp.where(pred, lo, mid + 1), jnp.where(pred, mid, hi)

    T, _ = lax.fori_loop(0, 16, body16, (lo0, hi0), unroll=4)
    T16 = (T - 32768).astype(jnp.int16)
    clt = (h16 < T16).astype(jnp.int16)
    n = clt.shape[0]
    while n > 1:
        n //= 2
        clt = clt[:n] + clt[n:]
    clt = clt.astype(jnp.int32)
    l15 = jnp.where(h16 == T16, (bits & 0x7FFF).astype(jnp.int16),
                    jnp.int16(32767))

    def body15(_, carry):
        lo, hi = carry
        mid = (lo + hi) >> 1
        x = (l15 <= mid.astype(jnp.int16)).astype(jnp.int16)
        n = x.shape[0]
        while n > 1:
            n //= 2
            x = x[:n] + x[n:]
        pred = clt + x.astype(jnp.int32) >= k
        return jnp.where(pred, lo, mid + 1), jnp.where(pred, mid, hi)

    lo, hi = lax.fori_loop(0, 15, body15,
                           (jnp.zeros((1, nt), jnp.int32),
                            jnp.full((1, nt), jnp.int32(32767))), unroll=4)
    thr = (T << 15) | lo
    return (bits <= thr).astype(jnp.float32)


_DN0 = (((0,), (0,)), ((), ()))


def _tfn_layer(src, tgt, tgtT, feats, Wc, bc, Wa, ba, Wb, bb, radius):
    dpx = src[:, 0:1] - tgtT[0:1, :]
    dpy = src[:, 1:2] - tgtT[1:2, :]
    dpz = src[:, 2:3] - tgtT[2:3, :]
    d2 = dpx * dpx + dpy * dpy + dpz * dpz
    mask = _kth_mask_T(d2, _K)
    dist = jnp.sqrt(d2 + 1e-12)
    inv = 1.0 / (dist + 1e-8)
    planes = []
    for s in range(_NSHELLS):
        c = s / (_NSHELLS - 1.0)
        g = jnp.exp(-_SCALE * (dist / radius - c) ** 2) * mask
        gi = g * inv
        planes.extend([g, gi * dpx, gi * dpy, gi * dpz])
    fb = feats.astype(jnp.bfloat16)
    cols = [lax.dot_general(p.astype(jnp.bfloat16), fb, _DN0,
                            preferred_element_type=jnp.float32)
            for p in planes]
    y = jnp.concatenate(cols, axis=1) * (1.0 / _K)
    y = jnp.maximum(jnp.dot(y, Wc, preferred_element_type=jnp.float32) + bc, 0.0)
    y = jnp.maximum(jnp.dot(y, Wa, preferred_element_type=jnp.float32) + ba, 0.0)
    y = jnp.maximum(jnp.dot(y, Wb, preferred_element_type=jnp.float32) + bb, 0.0)
    return y


def _body(x_ref, p1_ref, p1T_ref, p2_ref, p2T_ref, p3_ref, p3T_ref,
          Wc0_ref, bc0_ref, Wm0a_ref, bm0a_ref, Wm0b_ref, bm0b_ref,
          Wc1_ref, bc1_ref, Wm1a_ref, bm1a_ref, Wm1b_ref, bm1b_ref,
          Wc2_ref, bc2_ref, Wm2a_ref, bm2a_ref, Wm2b_ref, bm2b_ref,
          Wfc1_ref, bfc1_ref, Wfc2_ref, bfc2_ref, Wout_ref, bout_ref,
          out_ref):
    x = x_ref[0]
    p1, p1T = p1_ref[0], p1T_ref[0]
    p2, p2T = p2_ref[0], p2T_ref[0]
    p3, p3T = p3_ref[0], p3T_ref[0]

    f = _tfn_layer(x, p1, p1T, x, Wc0_ref[...], bc0_ref[...],
                   Wm0a_ref[...], bm0a_ref[...], Wm0b_ref[...], bm0b_ref[...],
                   _RADII[0])
    f = _tfn_layer(p1, p2, p2T, f, Wc1_ref[...], bc1_ref[...],
                   Wm1a_ref[...], bm1a_ref[...], Wm1b_ref[...], bm1b_ref[...],
                   _RADII[1])
    f = _tfn_layer(p2, p3, p3T, f, Wc2_ref[...], bc2_ref[...],
                   Wm2a_ref[...], bm2a_ref[...], Wm2b_ref[...], bm2b_ref[...],
                   _RADII[2])
    g = jnp.max(f, axis=0, keepdims=True)
    h = jnp.maximum(jnp.dot(g, Wfc1_ref[...], preferred_element_type=jnp.float32) + bfc1_ref[...], 0.0)
    h = jnp.maximum(jnp.dot(h, Wfc2_ref[...], preferred_element_type=jnp.float32) + bfc2_ref[...], 0.0)
    logits = jnp.dot(h, Wout_ref[...], preferred_element_type=jnp.float32) + bout_ref[...]
    m = jnp.max(logits, axis=-1, keepdims=True)
    e = jnp.exp(logits - m)
    out_ref[0] = e / jnp.sum(e, axis=-1, keepdims=True)


@jax.jit
def kernel(x, Wc0, bc0, Wm0a, bm0a, Wm0b, bm0b,
           Wc1, bc1, Wm1a, bm1a, Wm1b, bm1b,
           Wc2, bc2, Wm2a, bm2a, Wm2b, bm2b,
           Wfc1, bfc1, Wfc2, bfc2, Wout, bout):
    B, N, _ = x.shape
    xT = jnp.swapaxes(x, 1, 2)
    p1 = x[:, ::4, :]
    p1T = xT[:, :, ::4]
    p2 = x[:, ::16, :]
    p2T = xT[:, :, ::16]
    p3 = x[:, ::64, :]
    p3T = xT[:, :, ::64]

    def pb(a):
        return pl.BlockSpec((1,) + a.shape[1:], lambda b: (b,) + (0,) * (a.ndim - 1))

    def full(a):
        return pl.BlockSpec(a.shape, lambda b: (0,) * a.ndim)

    r2 = lambda b: b.reshape(1, -1)
    weights = (Wc0, r2(bc0), Wm0a, r2(bm0a), Wm0b, r2(bm0b),
               Wc1, r2(bc1), Wm1a, r2(bm1a), Wm1b, r2(bm1b),
               Wc2, r2(bc2), Wm2a, r2(bm2a), Wm2b, r2(bm2b),
               Wfc1, r2(bfc1), Wfc2, r2(bfc2), Wout, r2(bout))
    operands = (x, p1, p1T, p2, p2T, p3, p3T) + weights
    in_specs = [pb(x), pb(p1), pb(p1T), pb(p2), pb(p2T), pb(p3), pb(p3T)] + \
               [full(w) for w in weights]

    out = pl.pallas_call(
        _body,
        grid=(B,),
        in_specs=in_specs,
        out_specs=pl.BlockSpec((1, 1, 40), lambda b: (b, 0, 0)),
        out_shape=jax.ShapeDtypeStruct((B, 1, 40), jnp.float32),
        compiler_params=pltpu.CompilerParams(
            dimension_semantics=("arbitrary",),
        ),
    )(*operands)
    return out.reshape(B, 40)


# speedup vs baseline: 1.2005x; 1.0162x over previous
import jax
import jax.numpy as jnp
from jax import lax
from jax.experimental import pallas as pl
from jax.experimental.pallas import tpu as pltpu

_K = 32
_NSHELLS = 3
_RADII = (0.2, 0.4, 0.8)
_SCALE = 0.69314718056 * _NSHELLS ** 2


def _kth_mask_T(d2T, k):
    bits = lax.bitcast_convert_type(d2T, jnp.int32)
    nt = d2T.shape[1]
    h16 = ((bits >> 15) - 32768).astype(jnp.int16)
    lo0 = jnp.zeros((1, nt), jnp.int32)
    hi0 = jnp.full((1, nt), jnp.int32(65535))

    def body16(_, carry):
        lo, hi = carry
        mid = (lo + hi) >> 1
        mid16 = (mid - 32768).astype(jnp.int16)
        x = (h16 <= mid16).astype(jnp.int16)
        n = x.shape[0]
        while n > 1:
            n //= 2
            x = x[:n] + x[n:]
        pred = x.astype(jnp.int32) >= k
        return jnp.where(pred, lo, mid + 1), jnp.where(pred, mid, hi)

    T, _ = lax.fori_loop(0, 16, body16, (lo0, hi0), unroll=4)
    T16 = (T - 32768).astype(jnp.int16)
    clt = (h16 < T16).astype(jnp.int16)
    n = clt.shape[0]
    while n > 1:
        n //= 2
        clt = clt[:n] + clt[n:]
    clt = clt.astype(jnp.int32)
    l15 = jnp.where(h16 == T16, (bits & 0x7FFF).astype(jnp.int16),
                    jnp.int16(32767))

    def body15(_, carry):
        lo, hi = carry
        mid = (lo + hi) >> 1
        x = (l15 <= mid.astype(jnp.int16)).astype(jnp.int16)
        n = x.shape[0]
        while n > 1:
            n //= 2
            x = x[:n] + x[n:]
        pred = clt + x.astype(jnp.int32) >= k
        return jnp.where(pred, lo, mid + 1), jnp.where(pred, mid, hi)

    lo, hi = lax.fori_loop(0, 15, body15,
                           (jnp.zeros((1, nt), jnp.int32),
                            jnp.full((1, nt), jnp.int32(32767))), unroll=4)
    thr = (T << 15) | lo
    return (bits <= thr).astype(jnp.float32)


_DN0 = (((0,), (0,)), ((), ()))


def _tfn_layer(src, tgtT, feats, Wc, bc, Wa, ba, Wb, bb, radius):
    dpx = src[:, 0:1] - tgtT[0:1, :]
    dpy = src[:, 1:2] - tgtT[1:2, :]
    dpz = src[:, 2:3] - tgtT[2:3, :]
    d2 = dpx * dpx + dpy * dpy + dpz * dpz
    mask = _kth_mask_T(d2, _K)
    dist = jnp.sqrt(d2 + 1e-12)
    inv = 1.0 / (dist + 1e-8)
    planes = []
    for s in range(_NSHELLS):
        c = s / (_NSHELLS - 1.0)
        g = jnp.exp(-_SCALE * (dist / radius - c) ** 2) * mask
        gi = g * inv
        planes.extend([g, gi * dpx, gi * dpy, gi * dpz])
    fb = feats.astype(jnp.bfloat16)
    cols = [lax.dot_general(p.astype(jnp.bfloat16), fb, _DN0,
                            preferred_element_type=jnp.float32)
            for p in planes]
    y = jnp.concatenate(cols, axis=1) * (1.0 / _K)
    y = jnp.maximum(jnp.dot(y, Wc, preferred_element_type=jnp.float32) + bc, 0.0)
    y = jnp.maximum(jnp.dot(y, Wa, preferred_element_type=jnp.float32) + ba, 0.0)
    y = jnp.maximum(jnp.dot(y, Wb, preferred_element_type=jnp.float32) + bb, 0.0)
    return y


def _body(x_ref, p1_ref, p1T_ref, p2_ref, p2T_ref, p3T_ref,
          Wc0_ref, bc0_ref, Wm0a_ref, bm0a_ref, Wm0b_ref, bm0b_ref,
          Wc1_ref, bc1_ref, Wm1a_ref, bm1a_ref, Wm1b_ref, bm1b_ref,
          Wc2_ref, bc2_ref, Wm2a_ref, bm2a_ref, Wm2b_ref, bm2b_ref,
          Wfc1_ref, bfc1_ref, Wfc2_ref, bfc2_ref, Wout_ref, bout_ref,
          out_ref):
    x = x_ref[0]
    p1, p1T = p1_ref[0], p1T_ref[0]
    p2, p2T = p2_ref[0], p2T_ref[0]
    p3T = p3T_ref[0]

    f = _tfn_layer(x, p1T, x, Wc0_ref[...], bc0_ref[...],
                   Wm0a_ref[...], bm0a_ref[...], Wm0b_ref[...], bm0b_ref[...],
                   _RADII[0])
    f = _tfn_layer(p1, p2T, f, Wc1_ref[...], bc1_ref[...],
                   Wm1a_ref[...], bm1a_ref[...], Wm1b_ref[...], bm1b_ref[...],
                   _RADII[1])
    f = _tfn_layer(p2, p3T, f, Wc2_ref[...], bc2_ref[...],
                   Wm2a_ref[...], bm2a_ref[...], Wm2b_ref[...], bm2b_ref[...],
                   _RADII[2])
    g = jnp.max(f, axis=0, keepdims=True)
    h = jnp.maximum(jnp.dot(g, Wfc1_ref[...], preferred_element_type=jnp.float32) + bfc1_ref[...], 0.0)
    h = jnp.maximum(jnp.dot(h, Wfc2_ref[...], preferred_element_type=jnp.float32) + bfc2_ref[...], 0.0)
    logits = jnp.dot(h, Wout_ref[...], preferred_element_type=jnp.float32) + bout_ref[...]
    m = jnp.max(logits, axis=-1, keepdims=True)
    e = jnp.exp(logits - m)
    out_ref[0] = e / jnp.sum(e, axis=-1, keepdims=True)


@jax.jit
def kernel(x, Wc0, bc0, Wm0a, bm0a, Wm0b, bm0b,
           Wc1, bc1, Wm1a, bm1a, Wm1b, bm1b,
           Wc2, bc2, Wm2a, bm2a, Wm2b, bm2b,
           Wfc1, bfc1, Wfc2, bfc2, Wout, bout):
    B, N, _ = x.shape
    xT = jnp.swapaxes(x, 1, 2)
    p1 = x[:, ::4, :]
    p1T = xT[:, :, ::4]
    p2 = x[:, ::16, :]
    p2T = xT[:, :, ::16]
    p3T = xT[:, :, ::64]

    def pb(a):
        return pl.BlockSpec((1,) + a.shape[1:], lambda b: (b,) + (0,) * (a.ndim - 1))

    def full(a):
        return pl.BlockSpec(a.shape, lambda b: (0,) * a.ndim)

    r2 = lambda b: b.reshape(1, -1)
    weights = (Wc0, r2(bc0), Wm0a, r2(bm0a), Wm0b, r2(bm0b),
               Wc1, r2(bc1), Wm1a, r2(bm1a), Wm1b, r2(bm1b),
               Wc2, r2(bc2), Wm2a, r2(bm2a), Wm2b, r2(bm2b),
               Wfc1, r2(bfc1), Wfc2, r2(bfc2), Wout, r2(bout))
    operands = (x, p1, p1T, p2, p2T, p3T) + weights
    in_specs = [pb(x), pb(p1), pb(p1T), pb(p2), pb(p2T), pb(p3T)] + \
               [full(w) for w in weights]

    out = pl.pallas_call(
        _body,
        grid=(B,),
        in_specs=in_specs,
        out_specs=pl.BlockSpec((1, 1, 40), lambda b: (b, 0, 0)),
        out_shape=jax.ShapeDtypeStruct((B, 1, 40), jnp.float32),
        compiler_params=pltpu.CompilerParams(
            dimension_semantics=("arbitrary",),
        ),
    )(*operands)
    return out.reshape(B, 40)
